# Initial kernel scaffold; baseline (speedup 1.0000x reference)
import jax
import jax.numpy as jnp
from jax.experimental import pallas as pl


def kernel(input_tensor, pos_table, val_table):
    raise NotImplementedError("write your pallas kernel here")


# speedup vs baseline: 127.8422x; 127.8422x over previous
import functools

import jax
import jax.numpy as jnp
from jax import lax
from jax.experimental import pallas as pl
from jax.experimental.pallas import tpu as pltpu
from jax.experimental.pallas import tpu_sc as plsc

B = 1024
F = 1000
BIN = 10
D = 16
NIDX = F * BIN
VPAD = 1008
MAX_NORM = 1.0

NC, NS, L = 2, 16, 16
NW = NC * NS
ROWS_PER_W = B // NW
VECS = NIDX // L
ZVECS = VPAD // L


def _histogram_sc(idx_flat):
    mesh = plsc.VectorSubcoreMesh(core_axis_name="c", subcore_axis_name="s")

    @functools.partial(
        pl.kernel,
        mesh=mesh,
        out_type=jax.ShapeDtypeStruct((B * VPAD,), jnp.float32),
        scratch_types=[
            pltpu.VMEM((NIDX,), jnp.int32),
            pltpu.VMEM((ROWS_PER_W * VPAD,), jnp.float32),
        ],
        compiler_params=pltpu.CompilerParams(needs_layout_passes=False),
    )
    def hist_kernel(idx_hbm, counts_hbm, idx_v, counts_v):
        wid = lax.axis_index("s") * NC + lax.axis_index("c")
        base = wid * ROWS_PER_W
        zeros = jnp.zeros((L,), jnp.float32)
        ones = jnp.ones((L,), jnp.float32)

        def zero_body(k, _):
            counts_v[pl.ds(k * L, L)] = zeros
            return _

        lax.fori_loop(0, ROWS_PER_W * ZVECS, zero_body, None, unroll=8)

        def row_body(r, _):
            pltpu.sync_copy(idx_hbm.at[base + r], idx_v)
            roff = jnp.full((L,), 0, jnp.int32) + r * VPAD

            def vec_body(j, _):
                iv = idx_v[pl.ds(j * L, L)]
                plsc.addupdate_scatter(counts_v, [roff + iv], ones)
                return _

            return lax.fori_loop(0, VECS, vec_body, _, unroll=8)

        lax.fori_loop(0, ROWS_PER_W, row_body, None)
        pltpu.sync_copy(
            counts_v, counts_hbm.at[pl.ds(base * VPAD, ROWS_PER_W * VPAD)]
        )

    return hist_kernel(idx_flat)


def _finish_tc(counts, pos_table, val_pad):
    def body(counts_ref, pos_ref, val_ref, out_ref):
        def renorm(t):
            n = jnp.sqrt(jnp.sum(t * t, axis=1, keepdims=True))
            return t * jnp.minimum(1.0, MAX_NORM / jnp.maximum(n, 1e-12))

        val_r = renorm(val_ref[...])
        pos_r = renorm(pos_ref[...])
        pos_mean = jnp.sum(pos_r, axis=0, keepdims=True) * (1.0 / F)
        s = jnp.dot(counts_ref[...], val_r, preferred_element_type=jnp.float32)
        out_ref[...] = s * (1.0 / F) + pos_mean

    return pl.pallas_call(
        body,
        out_shape=jax.ShapeDtypeStruct((B, D), jnp.float32),
    )(counts, pos_table, val_pad)


def kernel(input_tensor, pos_table, val_table):
    idx_flat = input_tensor.reshape(B, NIDX)
    counts = _histogram_sc(idx_flat).reshape(B, VPAD)
    val_pad = jnp.pad(val_table, ((0, VPAD - F), (0, 0)))
    return _finish_tc(counts, pos_table, val_pad)
